# baseline (speedup 1.0000x reference)
import functools
import math

import jax
import jax.numpy as jnp
from jax.experimental import pallas as pl


def _body(x_ref, m1_ref, c1_ref, m2_ref, c2_ref, tbl_ref, ones_ref,
          gamma_ref, beta_ref, w2t_ref, b2_ref, out_ref):
    x = x_ref[0]
    y1 = jnp.dot(x, m1_ref[...], preferred_element_type=jnp.float32) + c1_ref[...]
    oh = (y1 == 0.0).astype(jnp.float32)
    emb = jnp.dot(oh, tbl_ref[...], preferred_element_type=jnp.float32)

    tc = jnp.dot(x, m2_ref[...], preferred_element_type=jnp.float32) + c2_ref[...]
    var = jnp.dot(tc * tc, ones_ref[...], preferred_element_type=jnp.float32)
    xhat = tc * jax.lax.rsqrt(var + 1e-5)
    t = xhat * gamma_ref[...] + beta_ref[...]
    g = t * (1.0 + jax.lax.erf(t * (1.0 / math.sqrt(2.0))))
    t2 = jnp.dot(g, w2t_ref[...], preferred_element_type=jnp.float32) + b2_ref[...]

    out_ref[:, 0:64] = emb
    out_ref[:, 64:128] = t2


def kernel(hour, weekday, time_sin_cos, hour_table, weekday_table,
           W1, b1, gamma, beta, W2, b2):
    B, L = hour.shape
    D4 = hour_table.shape[1]
    D2 = 2 * D4
    rows = B * L
    bb = next(c for c in (8192, 6400, 4096, 3200, 2048, 1600, 1024, 800,
                          512, 400, 256, 128, 64, 32, 16, 8)
              if rows % c == 0)
    grid = rows // bb

    x = jnp.concatenate(
        [hour.astype(jnp.float32)[..., None],
         weekday.astype(jnp.float32)[..., None],
         time_sin_cos.astype(jnp.float32)],
        axis=-1).reshape(grid, bb, 4)

    m1 = jnp.zeros((4, 32), jnp.float32)
    m1 = m1.at[0, 0:24].set(1.0)
    m1 = m1.at[1, 24:32].set(1.0)
    j = jnp.arange(32)
    c1 = jnp.where(j < 24, -j, -(j - 24)).astype(jnp.float32).reshape(1, 32)

    tbl = jnp.zeros((32, D2), jnp.float32)
    tbl = tbl.at[0:24, 0:D4].set(hour_table)
    tbl = tbl.at[24:31, D4:D2].set(weekday_table)

    w1t = W1.T.astype(jnp.float32)
    w1c = w1t - jnp.mean(w1t, axis=1, keepdims=True)
    b1c = (b1 - jnp.mean(b1)).astype(jnp.float32)
    m2 = jnp.zeros((4, D2), jnp.float32)
    m2 = m2.at[2:4, :].set(w1c)
    c2 = b1c.reshape(1, D2)

    ones = jnp.full((D2, D2), 1.0 / D2, jnp.float32)
    w2t = W2.T.astype(jnp.float32) * 0.5
    gammar = gamma.reshape(1, D2).astype(jnp.float32)
    betar = beta.reshape(1, D2).astype(jnp.float32)
    b2r = b2.reshape(1, D2).astype(jnp.float32)

    full = lambda shape: pl.BlockSpec(shape, lambda i: tuple(0 for _ in shape))
    out = pl.pallas_call(
        _body,
        grid=(grid,),
        in_specs=[
            pl.BlockSpec((1, bb, 4), lambda i: (i, 0, 0)),
            full((4, 32)),
            full((1, 32)),
            full((4, D2)),
            full((1, D2)),
            full((32, D2)),
            full((D2, D2)),
            full((1, D2)),
            full((1, D2)),
            full((D2, D2)),
            full((1, D2)),
        ],
        out_specs=pl.BlockSpec((bb, 2 * D2), lambda i: (i, 0)),
        out_shape=jax.ShapeDtypeStruct((rows, 2 * D2), jnp.float32),
    )(x, m1, c1, m2, c2, tbl, ones, gammar, betar, w2t, b2r)

    return out.reshape(B, L, 2 * D2)


# speedup vs baseline: 1.2837x; 1.2837x over previous
import functools
import math

import jax
import jax.numpy as jnp
from jax.experimental import pallas as pl


def _body(hw_ref, ts_ref, m1_ref, c1_ref, m2_ref, c2_ref, tbl_ref, ones_ref,
          gamma_ref, beta_ref, w2t_ref, b2_ref, out_ref):
    hw = hw_ref[0]
    y1 = jnp.dot(hw, m1_ref[...], preferred_element_type=jnp.float32) + c1_ref[...]
    oh = (y1 == 0.0).astype(jnp.float32)
    emb = jnp.dot(oh, tbl_ref[...], preferred_element_type=jnp.float32)

    ts = ts_ref[0]
    tc = jnp.dot(ts, m2_ref[...], preferred_element_type=jnp.float32) + c2_ref[...]
    var = jnp.dot(tc * tc, ones_ref[...], preferred_element_type=jnp.float32)
    xhat = tc * jax.lax.rsqrt(var + 1e-5)
    t = xhat * gamma_ref[...] + beta_ref[...]
    g = t * (1.0 + jax.lax.erf(t * (1.0 / math.sqrt(2.0))))
    t2 = jnp.dot(g, w2t_ref[...], preferred_element_type=jnp.float32) + b2_ref[...]

    out_ref[:, 0:64] = emb
    out_ref[:, 64:128] = t2


def kernel(hour, weekday, time_sin_cos, hour_table, weekday_table,
           W1, b1, gamma, beta, W2, b2):
    B, L = hour.shape
    D4 = hour_table.shape[1]
    D2 = 2 * D4
    rows = B * L
    bb = next(c for c in (8192, 6400, 4096, 3200, 2048, 1600, 1024, 800,
                          512, 400, 256, 128, 64, 32, 16, 8)
              if rows % c == 0)
    grid = rows // bb

    hw = jnp.stack([hour, weekday], axis=-1).astype(jnp.float32).reshape(grid, bb, 2)
    ts = time_sin_cos.astype(jnp.float32).reshape(grid, bb, 2)

    m1 = jnp.zeros((2, 32), jnp.float32)
    m1 = m1.at[0, 0:24].set(1.0)
    m1 = m1.at[1, 24:32].set(1.0)
    j = jnp.arange(32)
    c1 = jnp.where(j < 24, -j, -(j - 24)).astype(jnp.float32).reshape(1, 32)

    tbl = jnp.zeros((32, D2), jnp.float32)
    tbl = tbl.at[0:24, 0:D4].set(hour_table)
    tbl = tbl.at[24:31, D4:D2].set(weekday_table)

    w1t = W1.T.astype(jnp.float32)
    m2 = w1t - jnp.mean(w1t, axis=1, keepdims=True)
    c2 = (b1 - jnp.mean(b1)).astype(jnp.float32).reshape(1, D2)

    ones = jnp.full((D2, D2), 1.0 / D2, jnp.float32)
    w2t = W2.T.astype(jnp.float32) * 0.5
    gammar = gamma.reshape(1, D2).astype(jnp.float32)
    betar = beta.reshape(1, D2).astype(jnp.float32)
    b2r = b2.reshape(1, D2).astype(jnp.float32)

    full = lambda shape: pl.BlockSpec(shape, lambda i: tuple(0 for _ in shape))
    out = pl.pallas_call(
        _body,
        grid=(grid,),
        in_specs=[
            pl.BlockSpec((1, bb, 2), lambda i: (i, 0, 0)),
            pl.BlockSpec((1, bb, 2), lambda i: (i, 0, 0)),
            full((2, 32)),
            full((1, 32)),
            full((2, D2)),
            full((1, D2)),
            full((32, D2)),
            full((D2, D2)),
            full((1, D2)),
            full((1, D2)),
            full((D2, D2)),
            full((1, D2)),
        ],
        out_specs=pl.BlockSpec((bb, 2 * D2), lambda i: (i, 0)),
        out_shape=jax.ShapeDtypeStruct((rows, 2 * D2), jnp.float32),
    )(hw, ts, m1, c1, m2, c2, tbl, ones, gammar, betar, w2t, b2r)

    return out.reshape(B, L, 2 * D2)


# speedup vs baseline: 1.2853x; 1.0012x over previous
import functools
import math

import jax
import jax.numpy as jnp
from jax.experimental import pallas as pl


def _body(hw_ref, ts_ref, m1_ref, c1_ref, m2_ref, c2_ref, ones_ref,
          gamma_ref, beta_ref, m3_ref, c3_ref, out_ref):
    hw = hw_ref[0]
    y1 = jnp.dot(hw, m1_ref[...], preferred_element_type=jnp.float32) + c1_ref[...]
    oh = (y1 == 0.0).astype(jnp.float32)

    ts = ts_ref[0]
    tc = jnp.dot(ts, m2_ref[...], preferred_element_type=jnp.float32) + c2_ref[...]
    var = jnp.dot(tc * tc, ones_ref[...], preferred_element_type=jnp.float32)
    xhat = tc * jax.lax.rsqrt(var + 1e-5)
    t = xhat * gamma_ref[...] + beta_ref[...]
    g = t * (1.0 + jax.lax.erf(t * (1.0 / math.sqrt(2.0))))

    ohg = jnp.concatenate([oh, g], axis=1)
    out_ref[...] = jnp.dot(ohg, m3_ref[...],
                           preferred_element_type=jnp.float32) + c3_ref[...]


def kernel(hour, weekday, time_sin_cos, hour_table, weekday_table,
           W1, b1, gamma, beta, W2, b2):
    B, L = hour.shape
    D4 = hour_table.shape[1]
    D2 = 2 * D4
    rows = B * L
    bb = next(c for c in (8192, 6400, 4096, 3200, 2048, 1600, 1024, 800,
                          512, 400, 256, 128, 64, 32, 16, 8)
              if rows % c == 0)
    grid = rows // bb

    hw = jnp.stack([hour, weekday], axis=-1).astype(jnp.float32).reshape(grid, bb, 2)
    ts = time_sin_cos.astype(jnp.float32).reshape(grid, bb, 2)

    m1 = jnp.zeros((2, 32), jnp.float32)
    m1 = m1.at[0, 0:24].set(1.0)
    m1 = m1.at[1, 24:32].set(1.0)
    j = jnp.arange(32)
    c1 = jnp.where(j < 24, -j, -(j - 24)).astype(jnp.float32).reshape(1, 32)

    tbl = jnp.zeros((32, D2), jnp.float32)
    tbl = tbl.at[0:24, 0:D4].set(hour_table)
    tbl = tbl.at[24:31, D4:D2].set(weekday_table)
    m3 = jnp.zeros((96, 2 * D2), jnp.float32)
    m3 = m3.at[0:32, 0:D2].set(tbl)
    m3 = m3.at[32:96, D2:2 * D2].set(W2.T.astype(jnp.float32) * 0.5)
    c3 = jnp.concatenate([jnp.zeros((D2,), jnp.float32),
                          b2.astype(jnp.float32)]).reshape(1, 2 * D2)

    w1t = W1.T.astype(jnp.float32)
    m2 = w1t - jnp.mean(w1t, axis=1, keepdims=True)
    c2 = (b1 - jnp.mean(b1)).astype(jnp.float32).reshape(1, D2)

    ones = jnp.full((D2, D2), 1.0 / D2, jnp.float32)
    gammar = gamma.reshape(1, D2).astype(jnp.float32)
    betar = beta.reshape(1, D2).astype(jnp.float32)

    full = lambda shape: pl.BlockSpec(shape, lambda i: tuple(0 for _ in shape))
    out = pl.pallas_call(
        _body,
        grid=(grid,),
        in_specs=[
            pl.BlockSpec((1, bb, 2), lambda i: (i, 0, 0)),
            pl.BlockSpec((1, bb, 2), lambda i: (i, 0, 0)),
            full((2, 32)),
            full((1, 32)),
            full((2, D2)),
            full((1, D2)),
            full((D2, D2)),
            full((1, D2)),
            full((1, D2)),
            full((96, 2 * D2)),
            full((1, 2 * D2)),
        ],
        out_specs=pl.BlockSpec((bb, 2 * D2), lambda i: (i, 0)),
        out_shape=jax.ShapeDtypeStruct((rows, 2 * D2), jnp.float32),
    )(hw, ts, m1, c1, m2, c2, ones, gammar, betar, m3, c3)

    return out.reshape(B, L, 2 * D2)


# speedup vs baseline: 2.3373x; 1.8184x over previous
import functools
import math

import jax
import jax.numpy as jnp
from jax.experimental import pallas as pl


def _body(hw_ref, ts_ref, m1_ref, c1_ref, m2_ref, c2_ref, ones_ref,
          gamma_ref, beta_ref, m3_ref, c3_ref, out_ref):
    hw = hw_ref[0]
    y1 = jnp.dot(hw, m1_ref[...], preferred_element_type=jnp.float32) + c1_ref[...]
    oh = (y1 == 0.0).astype(jnp.float32)

    ts = ts_ref[0]
    tc = jnp.dot(ts, m2_ref[...], preferred_element_type=jnp.float32) + c2_ref[...]
    var = jnp.dot(tc * tc, ones_ref[...], preferred_element_type=jnp.float32)
    xhat = tc * jax.lax.rsqrt(var + 1e-5)
    t = xhat * gamma_ref[...] + beta_ref[...]
    g = t * (1.0 + jax.lax.erf(t * (1.0 / math.sqrt(2.0))))

    ohg = jnp.concatenate([oh, g], axis=1)
    res = jnp.dot(ohg, m3_ref[...],
                  preferred_element_type=jnp.float32) + c3_ref[...]
    bs = out_ref.shape[0]
    out_ref[...] = res.reshape(bs, out_ref.shape[1], out_ref.shape[2])


def kernel(hour, weekday, time_sin_cos, hour_table, weekday_table,
           W1, b1, gamma, beta, W2, b2):
    B, L = hour.shape
    D4 = hour_table.shape[1]
    D2 = 2 * D4
    rows = B * L
    bs = next(c for c in (128, 64, 32, 16, 8, 4, 2, 1) if B % c == 0)
    bb = bs * L
    grid = B // bs

    hw = jnp.stack([hour, weekday], axis=-1).astype(jnp.float32).reshape(grid, bb, 2)
    ts = time_sin_cos.astype(jnp.float32).reshape(grid, bb, 2)

    m1 = jnp.zeros((2, 32), jnp.float32)
    m1 = m1.at[0, 0:24].set(1.0)
    m1 = m1.at[1, 24:32].set(1.0)
    j = jnp.arange(32)
    c1 = jnp.where(j < 24, -j, -(j - 24)).astype(jnp.float32).reshape(1, 32)

    tbl = jnp.zeros((32, D2), jnp.float32)
    tbl = tbl.at[0:24, 0:D4].set(hour_table)
    tbl = tbl.at[24:31, D4:D2].set(weekday_table)
    m3 = jnp.zeros((96, 2 * D2), jnp.float32)
    m3 = m3.at[0:32, 0:D2].set(tbl)
    m3 = m3.at[32:96, D2:2 * D2].set(W2.T.astype(jnp.float32) * 0.5)
    c3 = jnp.concatenate([jnp.zeros((D2,), jnp.float32),
                          b2.astype(jnp.float32)]).reshape(1, 2 * D2)

    w1t = W1.T.astype(jnp.float32)
    m2 = w1t - jnp.mean(w1t, axis=1, keepdims=True)
    c2 = (b1 - jnp.mean(b1)).astype(jnp.float32).reshape(1, D2)

    ones = jnp.full((D2, D2), 1.0 / D2, jnp.float32)
    gammar = gamma.reshape(1, D2).astype(jnp.float32)
    betar = beta.reshape(1, D2).astype(jnp.float32)

    full = lambda shape: pl.BlockSpec(shape, lambda i: tuple(0 for _ in shape))
    out = pl.pallas_call(
        _body,
        grid=(grid,),
        in_specs=[
            pl.BlockSpec((1, bb, 2), lambda i: (i, 0, 0)),
            pl.BlockSpec((1, bb, 2), lambda i: (i, 0, 0)),
            full((2, 32)),
            full((1, 32)),
            full((2, D2)),
            full((1, D2)),
            full((D2, D2)),
            full((1, D2)),
            full((1, D2)),
            full((96, 2 * D2)),
            full((1, 2 * D2)),
        ],
        out_specs=pl.BlockSpec((bs, L, 2 * D2), lambda i: (i, 0, 0)),
        out_shape=jax.ShapeDtypeStruct((B, L, 2 * D2), jnp.float32),
    )(hw, ts, m1, c1, m2, c2, ones, gammar, betar, m3, c3)

    return out


# speedup vs baseline: 3.0115x; 1.2885x over previous
import math

import jax
import jax.numpy as jnp
from jax.experimental import pallas as pl


def _body(hw_ref, ts_ref, m1_ref, c1_ref, m2_ref, c2_ref, ones_ref,
          gamma_ref, beta_ref, m3_ref, c3_ref, out_ref):
    bs, L, D = out_ref.shape
    hw = hw_ref[0].T
    tsp = ts_ref[0].T

    y1 = jnp.dot(hw, m1_ref[...], preferred_element_type=jnp.float32) + c1_ref[...]
    oh = (y1 == 0.0).astype(jnp.float32)

    tc = jnp.dot(tsp, m2_ref[...], preferred_element_type=jnp.float32) + c2_ref[...]
    var = jnp.dot(tc * tc, ones_ref[...], preferred_element_type=jnp.float32)
    xhat = tc * jax.lax.rsqrt(var + 1e-5)
    t = xhat * gamma_ref[...] + beta_ref[...]
    g = t * (1.0 + jax.lax.erf(t * (1.0 / math.sqrt(2.0))))

    ohg = jnp.concatenate([oh, g], axis=1)
    res = jnp.dot(ohg, m3_ref[...],
                  preferred_element_type=jnp.float32) + c3_ref[...]
    out_ref[...] = res.reshape(bs, L, D)


def kernel(hour, weekday, time_sin_cos, hour_table, weekday_table,
           W1, b1, gamma, beta, W2, b2):
    B, L = hour.shape
    D4 = hour_table.shape[1]
    D2 = 2 * D4
    bs = next(c for c in (128, 64, 32, 16, 8, 4, 2, 1) if B % c == 0)
    bb = bs * L
    grid = B // bs

    hw = jnp.concatenate(
        [hour.astype(jnp.float32).reshape(grid, 1, bb),
         weekday.astype(jnp.float32).reshape(grid, 1, bb)], axis=1)
    t3 = time_sin_cos.astype(jnp.float32).reshape(grid, bb, 2)
    ts = jnp.moveaxis(t3, 2, 1)

    m1 = jnp.zeros((2, 32), jnp.float32)
    m1 = m1.at[0, 0:24].set(1.0)
    m1 = m1.at[1, 24:32].set(1.0)
    j = jnp.arange(32)
    c1 = jnp.where(j < 24, -j, -(j - 24)).astype(jnp.float32).reshape(1, 32)

    tbl = jnp.zeros((32, D2), jnp.float32)
    tbl = tbl.at[0:24, 0:D4].set(hour_table)
    tbl = tbl.at[24:31, D4:D2].set(weekday_table)
    m3 = jnp.zeros((96, 2 * D2), jnp.float32)
    m3 = m3.at[0:32, 0:D2].set(tbl)
    m3 = m3.at[32:96, D2:2 * D2].set(W2.T.astype(jnp.float32) * 0.5)
    c3 = jnp.concatenate([jnp.zeros((D2,), jnp.float32),
                          b2.astype(jnp.float32)]).reshape(1, 2 * D2)

    w1t = W1.T.astype(jnp.float32)
    m2 = w1t - jnp.mean(w1t, axis=1, keepdims=True)
    c2 = (b1 - jnp.mean(b1)).astype(jnp.float32).reshape(1, D2)

    ones = jnp.full((D2, D2), 1.0 / D2, jnp.float32)
    gammar = gamma.reshape(1, D2).astype(jnp.float32)
    betar = beta.reshape(1, D2).astype(jnp.float32)

    full = lambda shape: pl.BlockSpec(shape, lambda i: tuple(0 for _ in shape))
    out = pl.pallas_call(
        _body,
        grid=(grid,),
        in_specs=[
            pl.BlockSpec((1, 2, bb), lambda i: (i, 0, 0)),
            pl.BlockSpec((1, 2, bb), lambda i: (i, 0, 0)),
            full((2, 32)),
            full((1, 32)),
            full((2, D2)),
            full((1, D2)),
            full((D2, D2)),
            full((1, D2)),
            full((1, D2)),
            full((96, 2 * D2)),
            full((1, 2 * D2)),
        ],
        out_specs=pl.BlockSpec((bs, L, 2 * D2), lambda i: (i, 0, 0)),
        out_shape=jax.ShapeDtypeStruct((B, L, 2 * D2), jnp.float32),
    )(hw, ts, m1, c1, m2, c2, ones, gammar, betar, m3, c3)

    return out


# speedup vs baseline: 3.5289x; 1.1718x over previous
import math

import jax
import jax.numpy as jnp
from jax.experimental import pallas as pl


def _body(x_ref, m_ref, zrow_ref, on_ref, gamma_ref, beta_ref, m3_ref,
          c3_ref, out_ref):
    bs, L, D = out_ref.shape
    x = x_ref[0].T
    y = jnp.dot(x, m_ref[...], preferred_element_type=jnp.float32)
    oh = (y == zrow_ref[...]).astype(jnp.float32)

    sq = y * y
    var = jnp.dot(sq, on_ref[...], preferred_element_type=jnp.float32)
    tn = y * jax.lax.rsqrt(var + 1e-5) * gamma_ref[...] + beta_ref[...]
    g = tn * (1.0 + jax.lax.erf(tn))

    res = jnp.dot(oh + g, m3_ref[...],
                  preferred_element_type=jnp.float32) + c3_ref[...]
    out_ref[...] = res.reshape(bs, L, D)


def kernel(hour, weekday, time_sin_cos, hour_table, weekday_table,
           W1, b1, gamma, beta, W2, b2):
    B, L = hour.shape
    D4 = hour_table.shape[1]
    D2 = 2 * D4
    bs = next(c for c in (128, 64, 32, 16, 8, 4, 2, 1) if B % c == 0)
    bb = bs * L
    grid = B // bs

    t3 = time_sin_cos.astype(jnp.float32).reshape(grid, bb, 2)
    x5 = jnp.concatenate(
        [hour.astype(jnp.float32).reshape(grid, 1, bb),
         weekday.astype(jnp.float32).reshape(grid, 1, bb),
         jnp.moveaxis(t3, 2, 1),
         jnp.ones((grid, 1, bb), jnp.float32)], axis=1)

    w1t = W1.T.astype(jnp.float32)
    w1c = w1t - jnp.mean(w1t, axis=1, keepdims=True)
    b1c = (b1 - jnp.mean(b1)).astype(jnp.float32)
    j = jnp.arange(32)
    c1 = jnp.where(j < 24, -j, -(j - 24)).astype(jnp.float32)
    m = jnp.zeros((5, 32 + D2), jnp.float32)
    m = m.at[0, 0:24].set(1.0)
    m = m.at[1, 24:32].set(1.0)
    m = m.at[2:4, 32:32 + D2].set(w1c)
    m = m.at[4, 0:32].set(c1)
    m = m.at[4, 32:32 + D2].set(b1c)

    zrow = jnp.concatenate([jnp.zeros((32,), jnp.float32),
                            jnp.full((D2,), jnp.nan, jnp.float32)]).reshape(1, 96)

    on = jnp.zeros((96, 96), jnp.float32)
    on = on.at[32:96, :].set(1.0 / D2)

    rs2 = 1.0 / math.sqrt(2.0)
    gamma96 = jnp.concatenate([jnp.zeros((32,), jnp.float32),
                               gamma.astype(jnp.float32) * rs2]).reshape(1, 96)
    beta96 = jnp.concatenate([jnp.zeros((32,), jnp.float32),
                              beta.astype(jnp.float32) * rs2]).reshape(1, 96)

    tbl = jnp.zeros((32, D2), jnp.float32)
    tbl = tbl.at[0:24, 0:D4].set(hour_table)
    tbl = tbl.at[24:31, D4:D2].set(weekday_table)
    m3 = jnp.zeros((96, 2 * D2), jnp.float32)
    m3 = m3.at[0:32, 0:D2].set(tbl)
    m3 = m3.at[32:96, D2:2 * D2].set(
        W2.T.astype(jnp.float32) * (0.5 * math.sqrt(2.0)))
    c3 = jnp.concatenate([jnp.zeros((D2,), jnp.float32),
                          b2.astype(jnp.float32)]).reshape(1, 2 * D2)

    full = lambda shape: pl.BlockSpec(shape, lambda i: tuple(0 for _ in shape))
    out = pl.pallas_call(
        _body,
        grid=(grid,),
        in_specs=[
            pl.BlockSpec((1, 5, bb), lambda i: (i, 0, 0)),
            full((5, 96)),
            full((1, 96)),
            full((96, 96)),
            full((1, 96)),
            full((1, 96)),
            full((96, 2 * D2)),
            full((1, 2 * D2)),
        ],
        out_specs=pl.BlockSpec((bs, L, 2 * D2), lambda i: (i, 0, 0)),
        out_shape=jax.ShapeDtypeStruct((B, L, 2 * D2), jnp.float32),
    )(x5, m, zrow, on, gamma96, beta96, m3, c3)

    return out


# speedup vs baseline: 3.6305x; 1.0288x over previous
import math

import jax
import jax.numpy as jnp
from jax.experimental import pallas as pl


def _body(x_ref, m_ref, zrow_ref, on_ref, gamma_ref, beta_ref, m3_ref,
          c3_ref, out_ref):
    bs, L, D = out_ref.shape
    x = x_ref[0].T
    y = jnp.dot(x, m_ref[...], preferred_element_type=jnp.float32)
    oh = (y == zrow_ref[...]).astype(jnp.float32)

    sq = y * y
    var = jnp.dot(sq, on_ref[...], preferred_element_type=jnp.float32)
    tn = y * jax.lax.rsqrt(var + 1e-5) * gamma_ref[...] + beta_ref[...]
    g = tn * (1.0 + jax.lax.erf(tn))

    res = jnp.dot(oh + g, m3_ref[...],
                  preferred_element_type=jnp.float32) + c3_ref[...]
    out_ref[...] = res.reshape(bs, L, D)


def kernel(hour, weekday, time_sin_cos, hour_table, weekday_table,
           W1, b1, gamma, beta, W2, b2):
    B, L = hour.shape
    D4 = hour_table.shape[1]
    D2 = 2 * D4
    bs = next(c for c in (256, 128, 64, 32, 16, 8, 4, 2, 1) if B % c == 0)
    bb = bs * L
    grid = B // bs

    t3 = time_sin_cos.astype(jnp.float32).reshape(grid, bb, 2)
    x5 = jnp.concatenate(
        [hour.astype(jnp.float32).reshape(grid, 1, bb),
         weekday.astype(jnp.float32).reshape(grid, 1, bb),
         jnp.moveaxis(t3, 2, 1),
         jnp.ones((grid, 1, bb), jnp.float32)], axis=1)

    w1t = W1.T.astype(jnp.float32)
    w1c = w1t - jnp.mean(w1t, axis=1, keepdims=True)
    b1c = (b1 - jnp.mean(b1)).astype(jnp.float32)
    j = jnp.arange(32)
    c1 = jnp.where(j < 24, -j, -(j - 24)).astype(jnp.float32)
    m = jnp.zeros((5, 32 + D2), jnp.float32)
    m = m.at[0, 0:24].set(1.0)
    m = m.at[1, 24:32].set(1.0)
    m = m.at[2:4, 32:32 + D2].set(w1c)
    m = m.at[4, 0:32].set(c1)
    m = m.at[4, 32:32 + D2].set(b1c)

    zrow = jnp.concatenate([jnp.zeros((32,), jnp.float32),
                            jnp.full((D2,), jnp.nan, jnp.float32)]).reshape(1, 96)

    on = jnp.zeros((96, 96), jnp.float32)
    on = on.at[32:96, :].set(1.0 / D2)

    rs2 = 1.0 / math.sqrt(2.0)
    gamma96 = jnp.concatenate([jnp.zeros((32,), jnp.float32),
                               gamma.astype(jnp.float32) * rs2]).reshape(1, 96)
    beta96 = jnp.concatenate([jnp.zeros((32,), jnp.float32),
                              beta.astype(jnp.float32) * rs2]).reshape(1, 96)

    tbl = jnp.zeros((32, D2), jnp.float32)
    tbl = tbl.at[0:24, 0:D4].set(hour_table)
    tbl = tbl.at[24:31, D4:D2].set(weekday_table)
    m3 = jnp.zeros((96, 2 * D2), jnp.float32)
    m3 = m3.at[0:32, 0:D2].set(tbl)
    m3 = m3.at[32:96, D2:2 * D2].set(
        W2.T.astype(jnp.float32) * (0.5 * math.sqrt(2.0)))
    c3 = jnp.concatenate([jnp.zeros((D2,), jnp.float32),
                          b2.astype(jnp.float32)]).reshape(1, 2 * D2)

    full = lambda shape: pl.BlockSpec(shape, lambda i: tuple(0 for _ in shape))
    out = pl.pallas_call(
        _body,
        grid=(grid,),
        in_specs=[
            pl.BlockSpec((1, 5, bb), lambda i: (i, 0, 0)),
            full((5, 96)),
            full((1, 96)),
            full((96, 96)),
            full((1, 96)),
            full((1, 96)),
            full((96, 2 * D2)),
            full((1, 2 * D2)),
        ],
        out_specs=pl.BlockSpec((bs, L, 2 * D2), lambda i: (i, 0, 0)),
        out_shape=jax.ShapeDtypeStruct((B, L, 2 * D2), jnp.float32),
    )(x5, m, zrow, on, gamma96, beta96, m3, c3)

    return out


# speedup vs baseline: 3.6408x; 1.0028x over previous
import math

import jax
import jax.numpy as jnp
from jax.experimental import pallas as pl


def _body(x_ref, m_ref, zrow_ref, on_ref, gamma_ref, beta_ref, m3_ref,
          c3_ref, out_ref):
    bs, L, D = out_ref.shape
    x = x_ref[0].T
    y = jnp.dot(x, m_ref[...], preferred_element_type=jnp.float32)
    oh = (y == zrow_ref[...]).astype(jnp.float32)

    sq = y * y
    var = jnp.dot(sq, on_ref[...], preferred_element_type=jnp.float32)
    tn = y * jax.lax.rsqrt(var + 1e-5) * gamma_ref[...] + beta_ref[...]
    g = tn * (1.0 + jax.lax.erf(tn))

    res = jnp.dot(oh + g, m3_ref[...],
                  preferred_element_type=jnp.float32) + c3_ref[...]
    out_ref[...] = res.reshape(bs, L, D)


def kernel(hour, weekday, time_sin_cos, hour_table, weekday_table,
           W1, b1, gamma, beta, W2, b2):
    B, L = hour.shape
    D4 = hour_table.shape[1]
    D2 = 2 * D4
    bs = next(c for c in (512, 256, 128, 64, 32, 16, 8, 4, 2, 1) if B % c == 0)
    bb = bs * L
    grid = B // bs

    t3 = time_sin_cos.astype(jnp.float32).reshape(grid, bb, 2)
    x5 = jnp.concatenate(
        [hour.astype(jnp.float32).reshape(grid, 1, bb),
         weekday.astype(jnp.float32).reshape(grid, 1, bb),
         jnp.moveaxis(t3, 2, 1),
         jnp.ones((grid, 1, bb), jnp.float32)], axis=1)

    w1t = W1.T.astype(jnp.float32)
    w1c = w1t - jnp.mean(w1t, axis=1, keepdims=True)
    b1c = (b1 - jnp.mean(b1)).astype(jnp.float32)
    j = jnp.arange(32)
    c1 = jnp.where(j < 24, -j, -(j - 24)).astype(jnp.float32)
    m = jnp.zeros((5, 32 + D2), jnp.float32)
    m = m.at[0, 0:24].set(1.0)
    m = m.at[1, 24:32].set(1.0)
    m = m.at[2:4, 32:32 + D2].set(w1c)
    m = m.at[4, 0:32].set(c1)
    m = m.at[4, 32:32 + D2].set(b1c)

    zrow = jnp.concatenate([jnp.zeros((32,), jnp.float32),
                            jnp.full((D2,), jnp.nan, jnp.float32)]).reshape(1, 96)

    on = jnp.zeros((96, 96), jnp.float32)
    on = on.at[32:96, :].set(1.0 / D2)

    rs2 = 1.0 / math.sqrt(2.0)
    gamma96 = jnp.concatenate([jnp.zeros((32,), jnp.float32),
                               gamma.astype(jnp.float32) * rs2]).reshape(1, 96)
    beta96 = jnp.concatenate([jnp.zeros((32,), jnp.float32),
                              beta.astype(jnp.float32) * rs2]).reshape(1, 96)

    tbl = jnp.zeros((32, D2), jnp.float32)
    tbl = tbl.at[0:24, 0:D4].set(hour_table)
    tbl = tbl.at[24:31, D4:D2].set(weekday_table)
    m3 = jnp.zeros((96, 2 * D2), jnp.float32)
    m3 = m3.at[0:32, 0:D2].set(tbl)
    m3 = m3.at[32:96, D2:2 * D2].set(
        W2.T.astype(jnp.float32) * (0.5 * math.sqrt(2.0)))
    c3 = jnp.concatenate([jnp.zeros((D2,), jnp.float32),
                          b2.astype(jnp.float32)]).reshape(1, 2 * D2)

    full = lambda shape: pl.BlockSpec(shape, lambda i: tuple(0 for _ in shape))
    out = pl.pallas_call(
        _body,
        grid=(grid,),
        in_specs=[
            pl.BlockSpec((1, 5, bb), lambda i: (i, 0, 0)),
            full((5, 96)),
            full((1, 96)),
            full((96, 96)),
            full((1, 96)),
            full((1, 96)),
            full((96, 2 * D2)),
            full((1, 2 * D2)),
        ],
        out_specs=pl.BlockSpec((bs, L, 2 * D2), lambda i: (i, 0, 0)),
        out_shape=jax.ShapeDtypeStruct((B, L, 2 * D2), jnp.float32),
    )(x5, m, zrow, on, gamma96, beta96, m3, c3)

    return out
